# baseline (speedup 1.0000x reference)
import jax
import jax.numpy as jnp
from jax import lax
from jax.experimental import pallas as pl
from jax.experimental.pallas import tpu as pltpu

_C = 96
_H = 128
_W = 128
_HW = _H * _W
_PAD = _W
_NCLS = 15
_NBOX = 60


def _body(x_hbm, wc_ref, bc_ref, wh_ref, bh_ref,
          cls_ref, bbox_ref, s, xbuf, ml, mr, sem):
    i = pl.program_id(0)
    n = pl.num_programs(0)
    slot = lax.rem(i, 2)
    nslot = lax.rem(i + 1, 2)

    @pl.when(i == 0)
    def _prologue():
        pltpu.make_async_copy(x_hbm.at[0], xbuf.at[0], sem.at[0]).start()

    @pl.when(i + 1 < n)
    def _prefetch():
        pltpu.make_async_copy(x_hbm.at[i + 1], xbuf.at[nslot],
                              sem.at[nslot]).start()

    pltpu.make_async_copy(x_hbm.at[i], xbuf.at[slot], sem.at[slot]).wait()

    @pl.when(i == 0)
    def _zero_pads():
        zpad = jnp.zeros((3 * _C, _PAD), jnp.bfloat16)
        s[:, 0:_PAD] = zpad
        s[:, _PAD + _HW:] = zpad
        col = lax.broadcasted_iota(jnp.int32, (_C, _HW), 1) & (_W - 1)
        ml[...] = (col != 0).astype(jnp.bfloat16)
        mr[...] = (col != _W - 1).astype(jnp.bfloat16)

    xb = xbuf[slot].astype(jnp.bfloat16).reshape(_C, _HW)

    s[0:_C, _PAD:_PAD + _HW] = xb
    s[_C:2 * _C, _PAD:_PAD + _HW] = jnp.roll(xb, 1, axis=1) * ml[...]
    s[2 * _C:3 * _C, _PAD:_PAD + _HW] = jnp.roll(xb, -1, axis=1) * mr[...]

    acc = None
    for dy in (-1, 0, 1):
        off = _PAD + dy * _W
        part = lax.dot_general(
            wc_ref[dy + 1], s[:, off:off + _HW], (((1,), (0,)), ((), ())),
            preferred_element_type=jnp.float32)
        acc = part if acc is None else acc + part
    h = jnp.maximum(acc + bc_ref[...], 0.0)
    hb = h.astype(jnp.bfloat16)

    y = lax.dot_general(wh_ref[...], hb, (((1,), (0,)), ((), ())),
                        preferred_element_type=jnp.float32) + bh_ref[...]
    cls_ref[0] = y[0:_NCLS].reshape(_NCLS, _H, _W)
    bbox_ref[0] = y[16:16 + _NBOX].reshape(_NBOX, _H, _W)


def kernel(x, W_conv, b_conv, W_cls, b_cls, W_bbox, b_bbox):
    n = x.shape[0]
    wt = jnp.transpose(W_conv, (2, 3, 0, 1))
    wc = jnp.concatenate([wt[:, 1], wt[:, 0], wt[:, 2]], axis=-1)
    wc = wc.astype(jnp.bfloat16)
    wh = jnp.zeros((80, _C), jnp.float32)
    wh = wh.at[0:_NCLS].set(W_cls.reshape(_NCLS, _C))
    wh = wh.at[16:16 + _NBOX].set(W_bbox.reshape(_NBOX, _C))
    wh = wh.astype(jnp.bfloat16)
    bc = b_conv.reshape(_C, 1)
    bh = jnp.zeros((80, 1), jnp.float32)
    bh = bh.at[0:_NCLS].set(b_cls.reshape(_NCLS, 1))
    bh = bh.at[16:16 + _NBOX].set(b_bbox.reshape(_NBOX, 1))

    logits, bbox = pl.pallas_call(
        _body,
        grid=(n,),
        in_specs=[
            pl.BlockSpec(memory_space=pl.ANY),
            pl.BlockSpec((3, _C, 3 * _C), lambda i: (0, 0, 0)),
            pl.BlockSpec((_C, 1), lambda i: (0, 0)),
            pl.BlockSpec((80, _C), lambda i: (0, 0)),
            pl.BlockSpec((80, 1), lambda i: (0, 0)),
        ],
        out_specs=[
            pl.BlockSpec((1, _NCLS, _H, _W), lambda i: (i, 0, 0, 0)),
            pl.BlockSpec((1, _NBOX, _H, _W), lambda i: (i, 0, 0, 0)),
        ],
        out_shape=[
            jax.ShapeDtypeStruct((n, _NCLS, _H, _W), jnp.float32),
            jax.ShapeDtypeStruct((n, _NBOX, _H, _W), jnp.float32),
        ],
        scratch_shapes=[
            pltpu.VMEM((3 * _C, _HW + 2 * _PAD), jnp.bfloat16),
            pltpu.VMEM((2, _C, _H, _W), jnp.float32),
            pltpu.VMEM((_C, _HW), jnp.bfloat16),
            pltpu.VMEM((_C, _HW), jnp.bfloat16),
            pltpu.SemaphoreType.DMA((2,)),
        ],
    )(x, wc, bc, wh, bh)

    return (logits, bbox)


# speedup vs baseline: 1.0187x; 1.0187x over previous
import jax
import jax.numpy as jnp
from jax import lax
from jax.experimental import pallas as pl
from jax.experimental.pallas import tpu as pltpu

_C = 96
_H = 128
_W = 128
_HW = _H * _W
_PAD = _W
_NCLS = 15
_NBOX = 60


def _body(x_hbm, wc_ref, bc_ref, wh_ref, bh_ref,
          cls_ref, bbox_ref, s, xbuf, sem):
    i = pl.program_id(0)
    n = pl.num_programs(0)
    slot = lax.rem(i, 2)
    nslot = lax.rem(i + 1, 2)

    @pl.when(i == 0)
    def _prologue():
        pltpu.make_async_copy(x_hbm.at[0], xbuf.at[0], sem.at[0]).start()

    @pl.when(i + 1 < n)
    def _prefetch():
        pltpu.make_async_copy(x_hbm.at[i + 1], xbuf.at[nslot],
                              sem.at[nslot]).start()

    pltpu.make_async_copy(x_hbm.at[i], xbuf.at[slot], sem.at[slot]).wait()

    @pl.when(i == 0)
    def _zero_pads():
        zpad = jnp.zeros((3 * _C, _PAD), jnp.bfloat16)
        s[:, 0:_PAD] = zpad
        s[:, _PAD + _HW:] = zpad

    xb = xbuf[slot].astype(jnp.bfloat16).reshape(_C, _HW)

    s[0:_C, _PAD:_PAD + _HW] = xb
    col = lax.broadcasted_iota(jnp.int32, (_C, _HW), 1) & (_W - 1)
    xl = jnp.where(col == 0, jnp.bfloat16(0), jnp.roll(xb, 1, axis=1))
    s[_C:2 * _C, _PAD:_PAD + _HW] = xl
    xr = jnp.where(col == _W - 1, jnp.bfloat16(0), jnp.roll(xb, -1, axis=1))
    s[2 * _C:3 * _C, _PAD:_PAD + _HW] = xr

    acc = None
    for dy in (-1, 0, 1):
        off = _PAD + dy * _W
        part = lax.dot_general(
            wc_ref[dy + 1], s[:, off:off + _HW], (((1,), (0,)), ((), ())),
            preferred_element_type=jnp.float32)
        acc = part if acc is None else acc + part
    h = jnp.maximum(acc + bc_ref[...], 0.0)
    hb = h.astype(jnp.bfloat16)

    y = lax.dot_general(wh_ref[...], hb, (((1,), (0,)), ((), ())),
                        preferred_element_type=jnp.float32) + bh_ref[...]
    cls_ref[0] = y[0:_NCLS].reshape(_NCLS, _H, _W)
    bbox_ref[0] = y[16:16 + _NBOX].reshape(_NBOX, _H, _W)


def kernel(x, W_conv, b_conv, W_cls, b_cls, W_bbox, b_bbox):
    n = x.shape[0]
    wt = jnp.transpose(W_conv, (2, 3, 0, 1))
    wc = jnp.concatenate([wt[:, 1], wt[:, 0], wt[:, 2]], axis=-1)
    wc = wc.astype(jnp.bfloat16)
    wh = jnp.zeros((80, _C), jnp.float32)
    wh = wh.at[0:_NCLS].set(W_cls.reshape(_NCLS, _C))
    wh = wh.at[16:16 + _NBOX].set(W_bbox.reshape(_NBOX, _C))
    wh = wh.astype(jnp.bfloat16)
    bc = b_conv.reshape(_C, 1)
    bh = jnp.zeros((80, 1), jnp.float32)
    bh = bh.at[0:_NCLS].set(b_cls.reshape(_NCLS, 1))
    bh = bh.at[16:16 + _NBOX].set(b_bbox.reshape(_NBOX, 1))

    logits, bbox = pl.pallas_call(
        _body,
        grid=(n,),
        in_specs=[
            pl.BlockSpec(memory_space=pl.ANY),
            pl.BlockSpec((3, _C, 3 * _C), lambda i: (0, 0, 0)),
            pl.BlockSpec((_C, 1), lambda i: (0, 0)),
            pl.BlockSpec((80, _C), lambda i: (0, 0)),
            pl.BlockSpec((80, 1), lambda i: (0, 0)),
        ],
        out_specs=[
            pl.BlockSpec((1, _NCLS, _H, _W), lambda i: (i, 0, 0, 0)),
            pl.BlockSpec((1, _NBOX, _H, _W), lambda i: (i, 0, 0, 0)),
        ],
        out_shape=[
            jax.ShapeDtypeStruct((n, _NCLS, _H, _W), jnp.float32),
            jax.ShapeDtypeStruct((n, _NBOX, _H, _W), jnp.float32),
        ],
        scratch_shapes=[
            pltpu.VMEM((3 * _C, _HW + 2 * _PAD), jnp.bfloat16),
            pltpu.VMEM((2, _C, _H, _W), jnp.float32),
            pltpu.SemaphoreType.DMA((2,)),
        ],
    )(x, wc, bc, wh, bh)

    return (logits, bbox)
